# baseline (speedup 1.0000x reference)
import functools

import jax
import jax.numpy as jnp
from jax import lax
from jax.experimental import pallas as pl
from jax.experimental.pallas import tpu as pltpu
from jax.experimental.pallas import tpu_sc as plsc

_BATCH = 16384
_HIST = 50
_D = 32
_NC = 2
_NS = 16
_NW = _NC * _NS
_BT = 128
_NBT = _BATCH // _BT
_BT_PER_W = _NBT // _NW
_HG = 10
_NHG = _HIST // _HG


@functools.partial(
    pl.kernel,
    out_type=jax.ShapeDtypeStruct((_HIST, _D // 8, _NBT, 8, _BT),
                                  jnp.float32),
    mesh=plsc.VectorSubcoreMesh(core_axis_name="c", subcore_axis_name="s"),
    scratch_types=[
        pltpu.VMEM((_HIST, _BT), jnp.int32),
        pltpu.VMEM((_HG * _BT, _D), jnp.float32),
        pltpu.VMEM((_HG, _D // 8, 8, _BT), jnp.float32),
        pltpu.SemaphoreType.DMA,
        pltpu.SemaphoreType.DMA,
    ],
    compiler_params=pltpu.CompilerParams(use_tc_tiling_on_sc=False, needs_layout_passes=False),
)
def _gather_kernel(idxt_hbm, table_hbm, out_hbm, idxt_v, rows_v,
                   tile_v, gsem, wsem):
    wid = lax.axis_index("s") * _NC + lax.axis_index("c")
    lane = lax.iota(jnp.int32, 16)
    lanes16 = [lane + 16 * k for k in range(_BT // 16)]

    wback = None
    for jb in range(_BT_PER_W):
        jbg = wid * _BT_PER_W + jb
        pltpu.sync_copy(idxt_hbm.at[:, pl.ds(jbg * _BT, _BT)], idxt_v)

        for hg in range(_NHG):
            h0 = hg * _HG
            gathers = []
            for hq in range(_HG):
                gathers.append(pltpu.async_copy(
                    table_hbm.at[idxt_v.at[h0 + hq, :]],
                    rows_v.at[pl.ds(hq * _BT, _BT), :], gsem))
            if wback is not None:
                wback.wait()
                wback = None
            for g in gathers:
                g.wait()

            @pl.loop(0, _HG * _D)
            def _x(t):
                hq = t // _D
                d = t - hq * _D
                bbase = jnp.broadcast_to(hq * _BT, (16,)).astype(jnp.int32)
                dim = jnp.broadcast_to(d, (16,)).astype(jnp.int32)
                i = d // 8
                r = d - i * 8
                for k in range(_BT // 16):
                    tile_v[hq, i, r, pl.ds(16 * k, 16)] = (
                        plsc.load_gather(rows_v,
                                         [bbase + lanes16[k], dim]))

            wback = pltpu.async_copy(
                tile_v, out_hbm.at[pl.ds(h0, _HG), :, jbg, :, :], wsem)
    wback.wait()


def kernel(input, table):
    out5 = _gather_kernel(input.astype(jnp.int32).T, table)
    return out5.transpose(2, 4, 0, 1, 3).reshape(_BATCH, _HIST, _D)


# speedup vs baseline: 1.1200x; 1.1200x over previous
import functools

import jax
import jax.numpy as jnp
from jax import lax
from jax.experimental import pallas as pl
from jax.experimental.pallas import tpu as pltpu
from jax.experimental.pallas import tpu_sc as plsc

_BATCH = 16384
_HIST = 50
_D = 32
_NC = 2
_NS = 16
_NW = _NC * _NS
_BT = 128
_NBT = _BATCH // _BT
_BT_PER_W = _NBT // _NW
_HG = 10
_NHG = _HIST // _HG


@functools.partial(
    pl.kernel,
    out_type=jax.ShapeDtypeStruct((_HIST, _D // 8, _NBT, 8, _BT),
                                  jnp.float32),
    mesh=plsc.VectorSubcoreMesh(core_axis_name="c", subcore_axis_name="s"),
    scratch_types=[
        pltpu.VMEM((_HIST, _BT), jnp.int32),
        pltpu.VMEM((_HG * _BT, _D), jnp.float32),
        pltpu.VMEM((_HG, _D // 8, 8, _BT), jnp.float32),
        pltpu.SemaphoreType.DMA,
        pltpu.SemaphoreType.DMA,
    ],
    compiler_params=pltpu.CompilerParams(use_tc_tiling_on_sc=False, needs_layout_passes=False),
)
def _gather_kernel(idxt_hbm, table_hbm, out_hbm, idxt_v, rows_v,
                   tile_v, gsem, wsem):
    wid = lax.axis_index("s") * _NC + lax.axis_index("c")
    lane = lax.iota(jnp.int32, 16)
    lanes16 = [lane + 16 * k for k in range(_BT // 16)]

    wback = None
    for jb in range(_BT_PER_W):
        jbg = wid * _BT_PER_W + jb
        pltpu.sync_copy(idxt_hbm.at[:, pl.ds(jbg * _BT, _BT)], idxt_v)

        for hg in range(_NHG):
            h0 = hg * _HG
            gathers = []
            for hq in range(_HG):
                gathers.append(pltpu.async_copy(
                    table_hbm.at[idxt_v.at[h0 + hq, :]],
                    rows_v.at[pl.ds(hq * _BT, _BT), :], gsem))
            if wback is not None:
                wback.wait()
                wback = None
            for g in gathers:
                g.wait()

            @pl.loop(0, _HG * _D)
            def _x(t):
                hq = t // _D
                d = t - hq * _D
                bbase = jnp.broadcast_to(hq * _BT, (16,)).astype(jnp.int32)
                dim = jnp.broadcast_to(d, (16,)).astype(jnp.int32)
                i = d // 8
                r = d - i * 8
                vals = [plsc.load_gather(rows_v, [bbase + lanes16[k], dim])
                        for k in range(_BT // 16)]
                for k in range(_BT // 16):
                    tile_v[hq, i, r, pl.ds(16 * k, 16)] = vals[k]

            wback = pltpu.async_copy(
                tile_v, out_hbm.at[pl.ds(h0, _HG), :, jbg, :, :], wsem)
    wback.wait()


def kernel(input, table):
    out5 = _gather_kernel(input.astype(jnp.int32).T, table)
    return out5.transpose(2, 4, 0, 1, 3).reshape(_BATCH, _HIST, _D)


# speedup vs baseline: 1.2103x; 1.0806x over previous
import functools

import jax
import jax.numpy as jnp
from jax import lax
from jax.experimental import pallas as pl
from jax.experimental.pallas import tpu as pltpu
from jax.experimental.pallas import tpu_sc as plsc

_BATCH = 16384
_HIST = 50
_D = 32
_NC = 2
_NS = 16
_NW = _NC * _NS
_BT = 128
_NBT = _BATCH // _BT
_BT_PER_W = _NBT // _NW
_HG = 10
_NHG = _HIST // _HG


@functools.partial(
    pl.kernel,
    out_type=jax.ShapeDtypeStruct((_HIST, _D // 8, _NBT, 8, _BT),
                                  jnp.float32),
    mesh=plsc.VectorSubcoreMesh(core_axis_name="c", subcore_axis_name="s"),
    scratch_types=[
        pltpu.VMEM((_HIST, _BT), jnp.int32),
        pltpu.VMEM((_HG * _BT, _D), jnp.float32),
        pltpu.VMEM((_HG, _D // 8, 8, _BT), jnp.float32),
        pltpu.SemaphoreType.DMA,
        pltpu.SemaphoreType.DMA,
    ],
    compiler_params=pltpu.CompilerParams(use_tc_tiling_on_sc=False, needs_layout_passes=False),
)
def _gather_kernel(idxt_hbm, table_hbm, out_hbm, idxt_v, rows_v,
                   tile_v, gsem, wsem):
    wid = lax.axis_index("s") * _NC + lax.axis_index("c")
    lane = lax.iota(jnp.int32, 16)
    lanes16 = [lane + 16 * k for k in range(_BT // 16)]

    wback = None
    for jb in range(_BT_PER_W):
        jbg = wid * _BT_PER_W + jb
        pltpu.sync_copy(idxt_hbm.at[:, pl.ds(jbg * _BT, _BT)], idxt_v)

        for hg in range(_NHG):
            h0 = hg * _HG
            gathers = []
            for hq in range(_HG):
                gathers.append(pltpu.async_copy(
                    table_hbm.at[idxt_v.at[h0 + hq, :]],
                    rows_v.at[pl.ds(hq * _BT, _BT), :], gsem))
            if wback is not None:
                wback.wait()
                wback = None
            for g in gathers:
                g.wait()

            @plsc.parallel_loop(0, _HG * _D, unroll=2)
            def _x(t):
                hq = t // _D
                d = t - hq * _D
                bbase = jnp.broadcast_to(hq * _BT, (16,)).astype(jnp.int32)
                dim = jnp.broadcast_to(d, (16,)).astype(jnp.int32)
                i = d // 8
                r = d - i * 8
                vals = [plsc.load_gather(rows_v, [bbase + lanes16[k], dim])
                        for k in range(_BT // 16)]
                for k in range(_BT // 16):
                    tile_v[hq, i, r, pl.ds(16 * k, 16)] = vals[k]

            wback = pltpu.async_copy(
                tile_v, out_hbm.at[pl.ds(h0, _HG), :, jbg, :, :], wsem)
    wback.wait()


def kernel(input, table):
    out5 = _gather_kernel(input.astype(jnp.int32).T, table)
    return out5.transpose(2, 4, 0, 1, 3).reshape(_BATCH, _HIST, _D)


# speedup vs baseline: 1.2107x; 1.0004x over previous
import functools

import jax
import jax.numpy as jnp
from jax import lax
from jax.experimental import pallas as pl
from jax.experimental.pallas import tpu as pltpu
from jax.experimental.pallas import tpu_sc as plsc

_BATCH = 16384
_HIST = 50
_D = 32
_NC = 2
_NS = 16
_NW = _NC * _NS
_BT = 128
_NBT = _BATCH // _BT
_BT_PER_W = _NBT // _NW
_HG = 10
_NHG = _HIST // _HG


@functools.partial(
    pl.kernel,
    out_type=jax.ShapeDtypeStruct((_HIST, _D // 8, _NBT, 8, _BT),
                                  jnp.float32),
    mesh=plsc.VectorSubcoreMesh(core_axis_name="c", subcore_axis_name="s"),
    scratch_types=[
        pltpu.VMEM((_HIST, _BT), jnp.int32),
        pltpu.VMEM((_HG * _BT, _D), jnp.float32),
        pltpu.VMEM((_HG, _D // 8, 8, _BT), jnp.float32),
        pltpu.SemaphoreType.DMA,
        pltpu.SemaphoreType.DMA,
    ],
    compiler_params=pltpu.CompilerParams(use_tc_tiling_on_sc=False, needs_layout_passes=False),
)
def _gather_kernel(idxt_hbm, table_hbm, out_hbm, idxt_v, rows_v,
                   tile_v, gsem, wsem):
    wid = lax.axis_index("s") * _NC + lax.axis_index("c")
    lane = lax.iota(jnp.int32, 16)
    lanes16 = [lane + 16 * k for k in range(_BT // 16)]

    wback = None
    for jb in range(_BT_PER_W):
        jbg = wid * _BT_PER_W + jb
        pltpu.sync_copy(idxt_hbm.at[:, pl.ds(jbg * _BT, _BT)], idxt_v)

        for hg in range(_NHG):
            h0 = hg * _HG
            gathers = []
            for hq in range(_HG):
                gathers.append(pltpu.async_copy(
                    table_hbm.at[idxt_v.at[h0 + hq, :]],
                    rows_v.at[pl.ds(hq * _BT, _BT), :], gsem))
            if wback is not None:
                wback.wait()
                wback = None
            for g in gathers:
                g.wait()

            @plsc.parallel_loop(0, _HG * _D, unroll=4)
            def _x(t):
                hq = t // _D
                d = t - hq * _D
                bbase = jnp.broadcast_to(hq * _BT, (16,)).astype(jnp.int32)
                dim = jnp.broadcast_to(d, (16,)).astype(jnp.int32)
                i = d // 8
                r = d - i * 8
                vals = [plsc.load_gather(rows_v, [bbase + lanes16[k], dim])
                        for k in range(_BT // 16)]
                for k in range(_BT // 16):
                    tile_v[hq, i, r, pl.ds(16 * k, 16)] = vals[k]

            wback = pltpu.async_copy(
                tile_v, out_hbm.at[pl.ds(h0, _HG), :, jbg, :, :], wsem)
    wback.wait()


def kernel(input, table):
    out5 = _gather_kernel(input.astype(jnp.int32).T, table)
    return out5.transpose(2, 4, 0, 1, 3).reshape(_BATCH, _HIST, _D)
